# Initial kernel scaffold; baseline (speedup 1.0000x reference)
import jax
import jax.numpy as jnp
from jax.experimental import pallas as pl


def kernel(data, labels, n, mu, S_diag, cluster_labels):
    raise NotImplementedError("write your pallas kernel here")


# speedup vs baseline: 3.4932x; 3.4932x over previous
import functools

import jax
import jax.numpy as jnp
from jax.experimental import pallas as pl
from jax.experimental.pallas import tpu as pltpu

B = 1024
D = 128
C = 2000
K = 10
BB = 128
NBLK = B // BB


def _first_argmax(x, axis):
    m = jnp.max(x, axis=axis, keepdims=True)
    idx = jax.lax.broadcasted_iota(jnp.int32, x.shape, axis)
    big = jnp.iinfo(jnp.int32).max
    return jnp.min(jnp.where(x == m, idx, big), axis=axis)


def _act_body(data_ref, labels_ref, n_ref, mu_ref, s_ref, cl_ref,
              scores_ref, pred_ref, clusters_ref, j_ref, count_ref,
              iv_ref, muiv_ref, t3_ref):
    i = pl.program_id(0)

    @pl.when(i == 0)
    def _init():
        var = s_ref[:] / jnp.maximum(n_ref[:], 1.0)[:, None] + 1e-6
        iv = 1.0 / var
        iv_ref[:] = iv
        muiv_ref[:] = mu_ref[:] * iv
        t3_ref[:] = jnp.sum(mu_ref[:] * mu_ref[:] * iv, axis=1)[None, :]
        count_ref[:] = jnp.zeros_like(count_ref)

    x = data_ref[:]
    dn = (((1,), (1,)), ((), ()))
    t1 = jax.lax.dot_general(x * x, iv_ref[:], dn,
                             preferred_element_type=jnp.float32)
    t2 = jax.lax.dot_general(x, muiv_ref[:], dn,
                             preferred_element_type=jnp.float32)
    d2 = jnp.maximum(t1 - 2.0 * t2 + t3_ref[:], 0.0)
    dmin = jnp.min(d2, axis=1, keepdims=True)
    g = jnp.exp(-0.5 * (d2 - dmin))

    cl_f = cl_ref[:].astype(jnp.float32)
    assign = _first_argmax(cl_ref[:], axis=1)
    lab = labels_ref[:]
    mask = (lab[:, None] == assign[None, :]).astype(jnp.float32)
    gm = g * mask
    j = _first_argmax(gm, axis=1)
    clusters = _first_argmax(g, axis=1)

    gn = g / (jnp.sum(g, axis=1, keepdims=True) + 1e-12)
    scores = jax.lax.dot_general(gn, cl_f, (((1,), (0,)), ((), ())),
                                 preferred_element_type=jnp.float32)
    pred = _first_argmax(scores, axis=1)

    onehot = (j[:, None] == jax.lax.broadcasted_iota(jnp.int32, (BB, C), 1))
    count_ref[:] += jnp.sum(onehot.astype(jnp.float32), axis=0)

    scores_ref[:] = scores
    pred_ref[:] = pred
    clusters_ref[:] = clusters
    j_ref[:] = j


def _activation(data, labels, n, mu, S_diag, cluster_labels):
    grid = (NBLK,)
    out_shapes = (
        jax.ShapeDtypeStruct((B, K), jnp.float32),
        jax.ShapeDtypeStruct((B,), jnp.int32),
        jax.ShapeDtypeStruct((B,), jnp.int32),
        jax.ShapeDtypeStruct((B,), jnp.int32),
        jax.ShapeDtypeStruct((C,), jnp.float32),
    )
    in_specs = [
        pl.BlockSpec((BB, D), lambda i: (i, 0)),
        pl.BlockSpec((BB,), lambda i: (i,)),
        pl.BlockSpec((C,), lambda i: (0,)),
        pl.BlockSpec((C, D), lambda i: (0, 0)),
        pl.BlockSpec((C, D), lambda i: (0, 0)),
        pl.BlockSpec((C, K), lambda i: (0, 0)),
    ]
    out_specs = (
        pl.BlockSpec((BB, K), lambda i: (i, 0)),
        pl.BlockSpec((BB,), lambda i: (i,)),
        pl.BlockSpec((BB,), lambda i: (i,)),
        pl.BlockSpec((BB,), lambda i: (i,)),
        pl.BlockSpec((C,), lambda i: (0,)),
    )
    scratch = [
        pltpu.VMEM((C, D), jnp.float32),
        pltpu.VMEM((C, D), jnp.float32),
        pltpu.VMEM((1, C), jnp.float32),
    ]
    return pl.pallas_call(
        _act_body, grid=grid, in_specs=in_specs, out_specs=out_specs,
        out_shape=out_shapes, scratch_shapes=scratch,
    )(data, labels, n, mu, S_diag, cluster_labels)


def _upd_body(data_ref, j_ref, count_ref, n_ref, mu_ref, s_ref,
              nnew_ref, munew_ref, snew_ref):
    i = pl.program_id(0)

    @pl.when(i == 0)
    def _init():
        nnew_ref[:] = n_ref[:] + count_ref[:]
        munew_ref[:] = mu_ref[:]
        snew_ref[:] = s_ref[:]

    x = data_ref[:]
    j = j_ref[:]
    p = (j[:, None] == jax.lax.broadcasted_iota(jnp.int32, (BB, C), 1)
         ).astype(jnp.float32)
    dn_row = (((1,), (0,)), ((), ()))
    dn_col = (((0,), (0,)), ((), ()))
    mu_j = jax.lax.dot_general(p, mu_ref[:], dn_row,
                               preferred_element_type=jnp.float32)
    n_j = jax.lax.dot_general(p, nnew_ref[:][:, None], dn_row,
                              preferred_element_type=jnp.float32)
    e = x - mu_j
    w = 1.0 / n_j
    munew_ref[:] += jax.lax.dot_general(p, e * w, dn_col,
                                        preferred_element_type=jnp.float32)
    snew_ref[:] += jax.lax.dot_general(p, e * e, dn_col,
                                       preferred_element_type=jnp.float32)


def _update(data, j, count, n, mu, S_diag):
    grid = (NBLK,)
    out_shapes = (
        jax.ShapeDtypeStruct((C,), jnp.float32),
        jax.ShapeDtypeStruct((C, D), jnp.float32),
        jax.ShapeDtypeStruct((C, D), jnp.float32),
    )
    in_specs = [
        pl.BlockSpec((BB, D), lambda i: (i, 0)),
        pl.BlockSpec((BB,), lambda i: (i,)),
        pl.BlockSpec((C,), lambda i: (0,)),
        pl.BlockSpec((C,), lambda i: (0,)),
        pl.BlockSpec((C, D), lambda i: (0, 0)),
        pl.BlockSpec((C, D), lambda i: (0, 0)),
    ]
    out_specs = (
        pl.BlockSpec((C,), lambda i: (0,)),
        pl.BlockSpec((C, D), lambda i: (0, 0)),
        pl.BlockSpec((C, D), lambda i: (0, 0)),
    )
    return pl.pallas_call(
        _upd_body, grid=grid, in_specs=in_specs, out_specs=out_specs,
        out_shape=out_shapes,
    )(data, j, count, n, mu, S_diag)


def kernel(data, labels, n, mu, S_diag, cluster_labels):
    scores, pred, clusters, j, count = _activation(
        data, labels, n, mu, S_diag, cluster_labels)
    n_new, mu_new, S_new = _update(data, j, count, n, mu, S_diag)
    return scores, pred, clusters, n_new, mu_new, S_new


# speedup vs baseline: 4.1632x; 1.1918x over previous
import jax
import jax.numpy as jnp
from jax.experimental import pallas as pl
from jax.experimental.pallas import tpu as pltpu

B = 1024
D = 128
C = 2000
CP = 2048
K = 10
BB = 256
NBLK = B // BB

_DN_T = (((1,), (1,)), ((), ()))
_DN_ROW = (((1,), (0,)), ((), ()))
_DN_COL = (((0,), (0,)), ((), ()))
_BIG = jnp.iinfo(jnp.int32).max


def _argmax_col(x):
    m = jnp.max(x, axis=1, keepdims=True)
    idx = jax.lax.broadcasted_iota(jnp.int32, x.shape, 1)
    return jnp.min(jnp.where(x == m, idx, _BIG), axis=1, keepdims=True)


def _body(data_ref, labels_ref, n_ref, mu_ref, s_ref, cl_ref,
          scores_ref, pred_ref, clusters_ref, nnew_ref, munew_ref, snew_ref,
          iv_ref, muiv_ref, t3_ref, assign_ref, count_ref, j_ref, nncol_ref):
    i = pl.program_id(0)

    @pl.when(i == 0)
    def _init():
        var = s_ref[:] / jnp.maximum(n_ref[:], 1.0)[:, None] + 1e-6
        iv = 1.0 / var
        iv_ref[:] = iv
        muiv_ref[:] = mu_ref[:] * iv
        t3_ref[:] = jnp.sum(mu_ref[:] * mu_ref[:] * iv, axis=1)[None, :]
        cl = cl_ref[:]
        m = jnp.max(cl, axis=1, keepdims=True)
        cidx = jax.lax.broadcasted_iota(jnp.int32, cl.shape, 1)
        assign_ref[:] = jnp.min(jnp.where(cl == m, cidx, _BIG), axis=1)[None, :]
        count_ref[:] = jnp.zeros_like(count_ref)

    @pl.when(i < NBLK)
    def _activation():
        b = i
        x = data_ref[:]
        t1 = jax.lax.dot_general(x * x, iv_ref[:], _DN_T,
                                 preferred_element_type=jnp.float32)
        t2 = jax.lax.dot_general(x, muiv_ref[:], _DN_T,
                                 preferred_element_type=jnp.float32)
        d2 = jnp.maximum(t1 - 2.0 * t2 + t3_ref[:], 0.0)
        dmin = jnp.min(d2, axis=1, keepdims=True)
        g = jnp.exp(-0.5 * (d2 - dmin))

        iota = jax.lax.broadcasted_iota(jnp.int32, (BB, CP), 1)
        cc = jnp.min(jnp.where(g == 1.0, iota, _BIG), axis=1, keepdims=True)
        mask = (labels_ref[:] == assign_ref[:]).astype(jnp.float32)
        jc = _argmax_col(g * mask)

        gn = g / (jnp.sum(g, axis=1, keepdims=True) + 1e-12)
        sc = jax.lax.dot_general(gn, cl_ref[:].astype(jnp.float32), _DN_ROW,
                                 preferred_element_type=jnp.float32)
        pc = _argmax_col(sc)

        onehot = (jc == iota).astype(jnp.float32)
        count_ref[:] += jnp.sum(onehot, axis=0)[None, :]
        j_ref[pl.ds(b * BB, BB), :] = jc
        scores_ref[pl.ds(b * BB, BB), :] = sc
        pred_ref[pl.ds(b * BB, BB)] = pc[:, 0]
        clusters_ref[pl.ds(b * BB, BB)] = cc[:, 0]

    @pl.when(i == NBLK)
    def _init2():
        nn = n_ref[:] + count_ref[0, :]
        nnew_ref[:] = nn
        nncol_ref[:] = nn[:, None]
        munew_ref[:] = mu_ref[:]
        snew_ref[:] = s_ref[:]

    @pl.when(i >= NBLK)
    def _update():
        b = i - NBLK
        x = data_ref[:]
        jc = j_ref[pl.ds(b * BB, BB), :]
        iota = jax.lax.broadcasted_iota(jnp.int32, (BB, CP), 1)
        p = (jc == iota).astype(jnp.float32)
        mu_j = jax.lax.dot_general(p, mu_ref[:], _DN_ROW,
                                   preferred_element_type=jnp.float32)
        n_j = jax.lax.dot_general(p, nncol_ref[:], _DN_ROW,
                                  preferred_element_type=jnp.float32)
        e = x - mu_j
        w = 1.0 / n_j
        munew_ref[:] += jax.lax.dot_general(p, e * w, _DN_COL,
                                            preferred_element_type=jnp.float32)
        snew_ref[:] += jax.lax.dot_general(p, e * e, _DN_COL,
                                           preferred_element_type=jnp.float32)


def kernel(data, labels, n, mu, S_diag, cluster_labels):
    pad = CP - C
    mu_p = jnp.pad(mu, ((0, pad), (0, 0)))
    s_p = jnp.pad(S_diag, ((0, pad), (0, 0)))
    n_p = jnp.pad(n, (0, pad), constant_values=1.0)
    cl_p = jnp.pad(cluster_labels, ((0, pad), (0, 0)))
    labels_col = labels[:, None]

    out_shapes = (
        jax.ShapeDtypeStruct((B, K), jnp.float32),
        jax.ShapeDtypeStruct((B,), jnp.int32),
        jax.ShapeDtypeStruct((B,), jnp.int32),
        jax.ShapeDtypeStruct((CP,), jnp.float32),
        jax.ShapeDtypeStruct((CP, D), jnp.float32),
        jax.ShapeDtypeStruct((CP, D), jnp.float32),
    )
    blk = lambda i: (jnp.where(i < NBLK, i, i - NBLK), 0)
    in_specs = [
        pl.BlockSpec((BB, D), blk),
        pl.BlockSpec((BB, 1), blk),
        pl.BlockSpec((CP,), lambda i: (0,)),
        pl.BlockSpec((CP, D), lambda i: (0, 0)),
        pl.BlockSpec((CP, D), lambda i: (0, 0)),
        pl.BlockSpec((CP, K), lambda i: (0, 0)),
    ]
    out_specs = (
        pl.BlockSpec((B, K), lambda i: (0, 0)),
        pl.BlockSpec((B,), lambda i: (0,)),
        pl.BlockSpec((B,), lambda i: (0,)),
        pl.BlockSpec((CP,), lambda i: (0,)),
        pl.BlockSpec((CP, D), lambda i: (0, 0)),
        pl.BlockSpec((CP, D), lambda i: (0, 0)),
    )
    scratch = [
        pltpu.VMEM((CP, D), jnp.float32),
        pltpu.VMEM((CP, D), jnp.float32),
        pltpu.VMEM((1, CP), jnp.float32),
        pltpu.VMEM((1, CP), jnp.int32),
        pltpu.VMEM((1, CP), jnp.float32),
        pltpu.VMEM((B, 1), jnp.int32),
        pltpu.VMEM((CP, 1), jnp.float32),
    ]
    scores, pred, clusters, n_new, mu_new, S_new = pl.pallas_call(
        _body, grid=(2 * NBLK,), in_specs=in_specs, out_specs=out_specs,
        out_shape=out_shapes, scratch_shapes=scratch,
    )(data, labels_col, n_p, mu_p, s_p, cl_p)
    return (scores, pred, clusters, n_new[:C], mu_new[:C], S_new[:C])


# speedup vs baseline: 4.3626x; 1.0479x over previous
import jax
import jax.numpy as jnp
from jax.experimental import pallas as pl
from jax.experimental.pallas import tpu as pltpu

B = 1024
D = 128
C = 2000
CP = 2048
K = 10
BB = 256
NBLK = B // BB
NCH = CP // 128

_DN_T = (((1,), (1,)), ((), ()))
_DN_ROW = (((1,), (0,)), ((), ()))
_DN_COL = (((0,), (0,)), ((), ()))
_BIG = jnp.iinfo(jnp.int32).max


def _fold_lanes(x, op):
    m = x[:, 0:128]
    for k in range(1, NCH):
        m = op(m, x[:, k * 128:(k + 1) * 128])
    return m


def _min_lanes(x):
    return jnp.min(_fold_lanes(x, jnp.minimum), axis=1, keepdims=True)


def _max_lanes(x):
    return jnp.max(_fold_lanes(x, jnp.maximum), axis=1, keepdims=True)


def _first_index_where(cond, iota):
    cand = None
    for k in range(NCH):
        sl = slice(k * 128, (k + 1) * 128)
        c = jnp.where(cond[:, sl], iota[:, sl], _BIG)
        cand = c if cand is None else jnp.minimum(cand, c)
    return jnp.min(cand, axis=1, keepdims=True)


def _body(data_ref, labels_ref, n_ref, mu_ref, s_ref, cl_ref,
          scores_ref, pred_ref, clusters_ref, nnew_ref, munew_ref, snew_ref,
          iv_ref, muiv_ref, t3_ref, assign_ref, claug_ref, count_ref,
          j_ref, muaug_ref):
    i = pl.program_id(0)

    @pl.when(i == 0)
    def _init():
        var = s_ref[:] / jnp.maximum(n_ref[:], 1.0)[:, None] + 1e-6
        iv = 1.0 / var
        iv_ref[:] = iv
        muiv_ref[:] = mu_ref[:] * iv
        t3_ref[:] = jnp.sum(mu_ref[:] * mu_ref[:] * iv, axis=1)[None, :]
        cl = cl_ref[:]
        m = jnp.max(cl, axis=1, keepdims=True)
        cidx = jax.lax.broadcasted_iota(jnp.int32, cl.shape, 1)
        assign_ref[:] = jnp.min(jnp.where(cl == m, cidx, _BIG), axis=1)[None, :]
        claug_ref[:] = jnp.concatenate(
            [cl.astype(jnp.float32), jnp.ones((CP, 1), jnp.float32)], axis=1)
        count_ref[:] = jnp.zeros_like(count_ref)

    @pl.when(i < NBLK)
    def _activation():
        b = i
        x = data_ref[:]
        t1 = jax.lax.dot_general(x * x, iv_ref[:], _DN_T,
                                 preferred_element_type=jnp.float32)
        t2 = jax.lax.dot_general(x, muiv_ref[:], _DN_T,
                                 preferred_element_type=jnp.float32)
        d2 = jnp.maximum(t1 - 2.0 * t2 + t3_ref[:], 0.0)
        dmin = _min_lanes(d2)
        g = jnp.exp(-0.5 * (d2 - dmin))

        iota = jax.lax.broadcasted_iota(jnp.int32, (BB, CP), 1)
        cc = _first_index_where(g == 1.0, iota)
        gm = jnp.where(labels_ref[:] == assign_ref[:], g, 0.0)
        mg = _max_lanes(gm)
        jc = _first_index_where(gm == mg, iota)

        sa = jax.lax.dot_general(g, claug_ref[:], _DN_ROW,
                                 preferred_element_type=jnp.float32)
        scores = sa[:, :K] / (sa[:, K:K + 1] + 1e-12)
        m = jnp.max(scores, axis=1, keepdims=True)
        kidx = jax.lax.broadcasted_iota(jnp.int32, scores.shape, 1)
        pc = jnp.min(jnp.where(scores == m, kidx, _BIG), axis=1, keepdims=True)

        onehot = (jc == iota).astype(jnp.float32)
        count_ref[:] += jax.lax.dot_general(
            jnp.ones((1, BB), jnp.float32), onehot, _DN_ROW,
            preferred_element_type=jnp.float32)
        j_ref[pl.ds(b * BB, BB), :] = jc
        scores_ref[pl.ds(b * BB, BB), :] = scores
        pred_ref[pl.ds(b * BB, BB)] = pc[:, 0]
        clusters_ref[pl.ds(b * BB, BB)] = cc[:, 0]

    @pl.when(i == NBLK)
    def _init2():
        nn = n_ref[:] + count_ref[0, :]
        nnew_ref[:] = nn
        muaug_ref[:, :D] = mu_ref[:]
        muaug_ref[:, D:] = jnp.broadcast_to(nn[:, None], (CP, D))
        munew_ref[:] = mu_ref[:]
        snew_ref[:] = s_ref[:]

    @pl.when(i >= NBLK)
    def _update():
        b = i - NBLK
        x = data_ref[:]
        jc = j_ref[pl.ds(b * BB, BB), :]
        iota = jax.lax.broadcasted_iota(jnp.int32, (BB, CP), 1)
        p = (jc == iota).astype(jnp.float32)
        gath = jax.lax.dot_general(p, muaug_ref[:], _DN_ROW,
                                   preferred_element_type=jnp.float32)
        e = x - gath[:, :D]
        w = 1.0 / gath[:, D:D + 1]
        upd = jnp.concatenate([e * w, e * e], axis=1)
        delta = jax.lax.dot_general(p, upd, _DN_COL,
                                    preferred_element_type=jnp.float32)
        munew_ref[:] += delta[:, :D]
        snew_ref[:] += delta[:, D:]


def kernel(data, labels, n, mu, S_diag, cluster_labels):
    pad = CP - C
    mu_p = jnp.pad(mu, ((0, pad), (0, 0)))
    s_p = jnp.pad(S_diag, ((0, pad), (0, 0)))
    n_p = jnp.pad(n, (0, pad), constant_values=1.0)
    cl_p = jnp.pad(cluster_labels, ((0, pad), (0, 0)))
    labels_col = labels[:, None]

    out_shapes = (
        jax.ShapeDtypeStruct((B, K), jnp.float32),
        jax.ShapeDtypeStruct((B,), jnp.int32),
        jax.ShapeDtypeStruct((B,), jnp.int32),
        jax.ShapeDtypeStruct((CP,), jnp.float32),
        jax.ShapeDtypeStruct((CP, D), jnp.float32),
        jax.ShapeDtypeStruct((CP, D), jnp.float32),
    )
    blk = lambda i: (jnp.where(i < NBLK, i, i - NBLK), 0)
    in_specs = [
        pl.BlockSpec((BB, D), blk),
        pl.BlockSpec((BB, 1), blk),
        pl.BlockSpec((CP,), lambda i: (0,)),
        pl.BlockSpec((CP, D), lambda i: (0, 0)),
        pl.BlockSpec((CP, D), lambda i: (0, 0)),
        pl.BlockSpec((CP, K), lambda i: (0, 0)),
    ]
    out_specs = (
        pl.BlockSpec((B, K), lambda i: (0, 0)),
        pl.BlockSpec((B,), lambda i: (0,)),
        pl.BlockSpec((B,), lambda i: (0,)),
        pl.BlockSpec((CP,), lambda i: (0,)),
        pl.BlockSpec((CP, D), lambda i: (0, 0)),
        pl.BlockSpec((CP, D), lambda i: (0, 0)),
    )
    scratch = [
        pltpu.VMEM((CP, D), jnp.float32),
        pltpu.VMEM((CP, D), jnp.float32),
        pltpu.VMEM((1, CP), jnp.float32),
        pltpu.VMEM((1, CP), jnp.int32),
        pltpu.VMEM((CP, K + 1), jnp.float32),
        pltpu.VMEM((1, CP), jnp.float32),
        pltpu.VMEM((B, 1), jnp.int32),
        pltpu.VMEM((CP, 2 * D), jnp.float32),
    ]
    scores, pred, clusters, n_new, mu_new, S_new = pl.pallas_call(
        _body, grid=(2 * NBLK,), in_specs=in_specs, out_specs=out_specs,
        out_shape=out_shapes, scratch_shapes=scratch,
    )(data, labels_col, n_p, mu_p, s_p, cl_p)
    return (scores, pred, clusters, n_new[:C], mu_new[:C], S_new[:C])
